# baseline (device time: 7628 ns/iter reference)
import jax
import jax.numpy as jnp
from jax import lax
from jax.experimental import pallas as pl
from jax.experimental.pallas import tpu as pltpu

N_DEV = 4


def _combine(v0, i0, v1, i1):
    take = (v1 > v0) | ((v1 == v0) & (i1 < i0))
    return jnp.where(take, v1, v0), jnp.where(take, i1, i0)


def kernel(x):
    m_per, n = x.shape

    def body(x_ref, out_ref, src_ref, recv_ref, send_sems, recv_sems):
        my_pos = lax.axis_index("i")
        partner0 = jnp.bitwise_xor(my_pos, 1)
        partner1 = (N_DEV - 1) - my_pos

        barrier_sem = pltpu.get_barrier_semaphore()
        for p in (partner0, partner1):
            pl.semaphore_signal(
                barrier_sem, inc=1,
                device_id=(p,), device_id_type=pl.DeviceIdType.MESH,
            )

        xv = x_ref[:, :]
        vmax = jnp.max(xv, axis=0, keepdims=True)
        row_ids = lax.broadcasted_iota(jnp.int32, (m_per, n), 0)
        local_idx = jnp.min(
            jnp.where(xv == vmax, row_ids, m_per), axis=0, keepdims=True
        ).astype(jnp.float32)
        src_ref[0, 0:1, :] = vmax
        src_ref[0, 1:2, :] = local_idx + my_pos.astype(jnp.float32) * float(m_per)

        pl.semaphore_wait(barrier_sem, 2)

        for s, partner in ((0, partner0), (1, partner1)):
            rdma = pltpu.make_async_remote_copy(
                src_ref=src_ref.at[s],
                dst_ref=recv_ref.at[s],
                send_sem=send_sems.at[s],
                recv_sem=recv_sems.at[s],
                device_id=(partner,),
                device_id_type=pl.DeviceIdType.MESH,
            )
            rdma.start()
            rdma.wait_recv()
            v, i = _combine(
                src_ref[s, 0:1, :], src_ref[s, 1:2, :],
                recv_ref[s, 0:1, :], recv_ref[s, 1:2, :],
            )
            if s == 0:
                src_ref[1, 0:1, :] = v
                src_ref[1, 1:2, :] = i
            else:
                out_ref[0:1, :] = v
                out_ref[1:2, :] = i
            rdma.wait_send()

    return pl.pallas_call(
        body,
        out_shape=jax.ShapeDtypeStruct((2, n), jnp.float32),
        in_specs=[pl.BlockSpec(memory_space=pltpu.VMEM)],
        out_specs=pl.BlockSpec(memory_space=pltpu.VMEM),
        scratch_shapes=[
            pltpu.VMEM((2, 2, n), jnp.float32),
            pltpu.VMEM((2, 2, n), jnp.float32),
            pltpu.SemaphoreType.DMA((2,)),
            pltpu.SemaphoreType.DMA((2,)),
        ],
        compiler_params=pltpu.CompilerParams(collective_id=0),
    )(x)


# device time: 6268 ns/iter; 1.2170x vs baseline; 1.2170x over previous
import jax
import jax.numpy as jnp
from jax import lax
from jax.experimental import pallas as pl
from jax.experimental.pallas import tpu as pltpu

N_DEV = 4


def kernel(x):
    m_per, n = x.shape

    def body(x_hbm, out_ref, x_vmem, copy_sem, stage_ref, comm_ref,
             send_sems, recv_sems):
        my_pos = lax.axis_index("i")

        barrier_sem = pltpu.get_barrier_semaphore()
        for r in range(1, N_DEV):
            pl.semaphore_signal(
                barrier_sem, inc=1,
                device_id=((my_pos + r) % N_DEV,),
                device_id_type=pl.DeviceIdType.MESH,
            )

        cp = pltpu.make_async_copy(x_hbm, x_vmem, copy_sem)
        cp.start()
        cp.wait()

        xv = x_vmem[:, :]
        vmax = jnp.max(xv, axis=0, keepdims=True)
        row_ids = lax.broadcasted_iota(jnp.int32, (m_per, n), 0)
        local_idx = jnp.min(
            jnp.where(xv == vmax, row_ids, m_per), axis=0, keepdims=True
        ).astype(jnp.float32)
        stage_ref[0:1, :] = vmax
        stage_ref[1:2, :] = local_idx + my_pos.astype(jnp.float32) * float(m_per)

        pl.semaphore_wait(barrier_sem, N_DEV - 1)

        sends = []
        for r in range(1, N_DEV):
            rdma = pltpu.make_async_remote_copy(
                src_ref=stage_ref,
                dst_ref=comm_ref.at[r],
                send_sem=send_sems.at[r - 1],
                recv_sem=recv_sems.at[r - 1],
                device_id=((my_pos - r) % N_DEV,),
                device_id_type=pl.DeviceIdType.MESH,
            )
            rdma.start()
            sends.append(rdma)

        for r in (1, 3, 2):
            recv = pltpu.make_async_remote_copy(
                src_ref=stage_ref,
                dst_ref=comm_ref.at[r],
                send_sem=send_sems.at[r - 1],
                recv_sem=recv_sems.at[r - 1],
                device_id=((my_pos + r) % N_DEV,),
                device_id_type=pl.DeviceIdType.MESH,
            )
            recv.wait_recv()

        best_v = stage_ref[0:1, :]
        best_i = stage_ref[1:2, :]
        for r in range(1, N_DEV):
            v = comm_ref[r, 0:1, :]
            i = comm_ref[r, 1:2, :]
            take = (v > best_v) | ((v == best_v) & (i < best_i))
            best_v = jnp.where(take, v, best_v)
            best_i = jnp.where(take, i, best_i)
        out_ref[0:1, :] = best_v
        out_ref[1:2, :] = best_i

        for rdma in sends:
            rdma.wait_send()

    return pl.pallas_call(
        body,
        out_shape=jax.ShapeDtypeStruct((2, n), jnp.float32),
        in_specs=[pl.BlockSpec(memory_space=pl.ANY)],
        out_specs=pl.BlockSpec(memory_space=pltpu.VMEM),
        scratch_shapes=[
            pltpu.VMEM((m_per, n), jnp.float32),
            pltpu.SemaphoreType.DMA,
            pltpu.VMEM((2, n), jnp.float32),
            pltpu.VMEM((N_DEV, 2, n), jnp.float32),
            pltpu.SemaphoreType.DMA((N_DEV - 1,)),
            pltpu.SemaphoreType.DMA((N_DEV - 1,)),
        ],
        compiler_params=pltpu.CompilerParams(collective_id=0),
    )(x)
